# baseline (speedup 1.0000x reference)
import functools

import jax
import jax.numpy as jnp
from jax import lax
from jax.experimental import pallas as pl
from jax.experimental.pallas import tpu as pltpu

_B = 64
_D = 128
_S = 3
_NB = 4000


def _body(starts_ref, lo_ref, hi_ref,
          x_ref, starts_v_ref, ends_v_ref, w_att_ref, b_att_ref, ctx_ref,
          w_out_ref, b_out_ref, w_s1_ref, b_s1_ref, w_s2_ref, b_s2_ref,
          out_ref,
          m_ref, s_ref, acc_ref, max_ref,
          *, nb, nblocks):
    pid = pl.program_id(0)

    @pl.when(pid == 0)
    def _init():
        m_ref[...] = jnp.full_like(m_ref[...], -jnp.inf)
        s_ref[...] = jnp.zeros_like(s_ref[...])
        acc_ref[...] = jnp.zeros_like(acc_ref[...])
        max_ref[...] = jnp.full_like(max_ref[...], -jnp.inf)

    x = x_ref[...]
    base = pid * nb
    ri = lax.broadcasted_iota(jnp.int32, (nb, _B), 0) + base
    starts_row = starts_v_ref[...]
    ends_row = ends_v_ref[...]
    onehot_b = (ri >= starts_row) & (ri < ends_row)
    onehot_f = onehot_b.astype(jnp.float32)

    logits = lax.dot_general(x, w_att_ref[...], (((1,), (1,)), ((), ())),
                             preferred_element_type=jnp.float32)
    logits = logits + b_att_ref[...]
    scores = lax.dot_general(jnp.tanh(logits), ctx_ref[...],
                             (((1,), (1,)), ((), ())),
                             preferred_element_type=jnp.float32)

    m_old = m_ref[...]
    m_blk = jnp.max(jnp.where(onehot_b, scores, -jnp.inf), axis=0,
                    keepdims=True)
    m_new = jnp.maximum(m_old, m_blk)
    scale = jnp.where(m_new == -jnp.inf, 0.0, jnp.exp(m_old - m_new))
    p = jnp.where(onehot_b, jnp.exp(scores - m_new), 0.0)

    m_ref[...] = m_new
    ones_row = jnp.ones((1, nb), dtype=jnp.float32)
    s_ref[...] = s_ref[...] * scale + lax.dot_general(
        ones_row, p, (((1,), (0,)), ((), ())),
        preferred_element_type=jnp.float32)
    q = jnp.concatenate([p, onehot_f], axis=1)
    scale2 = jnp.concatenate([scale, jnp.ones_like(scale)], axis=1)
    acc_ref[...] = acc_ref[...] * scale2 + lax.dot_general(
        x, q, (((0,), (0,)), ((), ())), preferred_element_type=jnp.float32)

    ri1 = lax.broadcasted_iota(jnp.int32, (nb, 1), 0) + base

    def seg_body(s, carry):
        st = starts_ref[s]
        en = starts_ref[s + 1]
        msk = (ri1 >= st) & (ri1 < en)
        colmax = jnp.max(jnp.where(msk, x, -jnp.inf), axis=0, keepdims=True)
        max_ref[pl.ds(s, 1), :] = jnp.maximum(max_ref[pl.ds(s, 1), :], colmax)
        return carry

    lax.fori_loop(lo_ref[pid], hi_ref[pid] + 1, seg_body, 0)

    @pl.when(pid == nblocks - 1)
    def _epilogue():
        counts = (ends_row - starts_row).astype(jnp.float32)
        acc = acc_ref[...]
        mean_t = acc[:, _B:] / jnp.maximum(counts, 1.0)
        graph_t = acc[:, :_B] / (s_ref[...] + 1e-8)
        attn = lax.dot_general(graph_t, w_out_ref[...],
                               (((0,), (1,)), ((), ())),
                               preferred_element_type=jnp.float32)
        attn = attn + b_out_ref[...]
        maxp = max_ref[...]
        w_s1 = w_s1_ref[...]
        h = lax.dot_general(mean_t, w_s1[:, 0:_D],
                            (((0,), (1,)), ((), ())),
                            preferred_element_type=jnp.float32)
        h = h + lax.dot_general(maxp, w_s1[:, _D:2 * _D],
                                (((1,), (1,)), ((), ())),
                                preferred_element_type=jnp.float32)
        h = h + lax.dot_general(attn, w_s1[:, 2 * _D:3 * _D],
                                (((1,), (1,)), ((), ())),
                                preferred_element_type=jnp.float32)
        h = jnp.maximum(h + b_s1_ref[...], 0.0)
        lg = lax.dot_general(h, w_s2_ref[...], (((1,), (1,)), ((), ())),
                             preferred_element_type=jnp.float32)
        lg = lg + b_s2_ref[...]
        lg = lg - jnp.max(lg, axis=1, keepdims=True)
        e = jnp.exp(lg)
        sel = e / jnp.sum(e, axis=1, keepdims=True)
        eye = (lax.broadcasted_iota(jnp.int32, (_D, _D), 0) ==
               lax.broadcasted_iota(jnp.int32, (_D, _D), 1)
               ).astype(jnp.float32)
        mean_bd = lax.dot_general(mean_t, eye, (((0,), (0,)), ((), ())),
                                  preferred_element_type=jnp.float32)
        out_ref[...] = (sel[:, 0:1] * mean_bd + sel[:, 1:2] * maxp +
                        sel[:, 2:3] * attn)


def kernel(x, batch, W_att, b_att, ctx, W_out, b_out, W_s1, b_s1, W_s2,
           b_s2):
    n, d = x.shape
    nb = _NB
    nblocks = n // nb
    assert n % nb == 0

    batch32 = batch.astype(jnp.int32)
    seg_ids = jnp.arange(_B + 1, dtype=jnp.int32)
    starts_ext = jnp.searchsorted(batch32, seg_ids, side="left").astype(
        jnp.int32)
    starts_v = starts_ext[:_B].reshape(1, _B)
    ends_v = starts_ext[1:].reshape(1, _B)
    blk0 = jnp.arange(nblocks, dtype=jnp.int32) * nb
    lo_arr = batch32[blk0]
    hi_arr = batch32[blk0 + (nb - 1)]

    body = functools.partial(_body, nb=nb, nblocks=nblocks)

    grid_spec = pltpu.PrefetchScalarGridSpec(
        num_scalar_prefetch=3,
        grid=(nblocks,),
        in_specs=[
            pl.BlockSpec((nb, d), lambda i, *_: (i, 0)),
            pl.BlockSpec((1, _B), lambda i, *_: (0, 0)),
            pl.BlockSpec((1, _B), lambda i, *_: (0, 0)),
            pl.BlockSpec(W_att.shape, lambda i, *_: (0, 0)),
            pl.BlockSpec((1, _B), lambda i, *_: (0, 0)),
            pl.BlockSpec((1, _B), lambda i, *_: (0, 0)),
            pl.BlockSpec(W_out.shape, lambda i, *_: (0, 0)),
            pl.BlockSpec((1, d), lambda i, *_: (0, 0)),
            pl.BlockSpec(W_s1.shape, lambda i, *_: (0, 0)),
            pl.BlockSpec((1, d), lambda i, *_: (0, 0)),
            pl.BlockSpec(W_s2.shape, lambda i, *_: (0, 0)),
            pl.BlockSpec((1, _S), lambda i, *_: (0, 0)),
        ],
        out_specs=pl.BlockSpec((_B, d), lambda i, *_: (0, 0)),
        scratch_shapes=[
            pltpu.VMEM((1, _B), jnp.float32),
            pltpu.VMEM((1, _B), jnp.float32),
            pltpu.VMEM((d, 2 * _B), jnp.float32),
            pltpu.VMEM((_B, d), jnp.float32),
        ],
    )

    return pl.pallas_call(
        body,
        grid_spec=grid_spec,
        out_shape=jax.ShapeDtypeStruct((_B, d), jnp.float32),
        compiler_params=pltpu.CompilerParams(
            dimension_semantics=("arbitrary",)),
    )(starts_ext, lo_arr, hi_arr,
      x, starts_v, ends_v, W_att, b_att.reshape(1, -1), ctx.reshape(1, -1),
      W_out, b_out.reshape(1, -1), W_s1, b_s1.reshape(1, -1), W_s2,
      b_s2.reshape(1, -1))


# speedup vs baseline: 1.0356x; 1.0356x over previous
import functools

import jax
import jax.numpy as jnp
from jax import lax
from jax.experimental import pallas as pl
from jax.experimental.pallas import tpu as pltpu

_B = 64
_D = 128
_S = 3
_NB = 2000


def _body(starts_ref, lo_ref, hi_ref,
          x_ref, starts_v_ref, ends_v_ref, w_att_ref, b_att_ref, ctx_ref,
          w_out_ref, b_out_ref, w_s1_ref, b_s1_ref, w_s2_ref, b_s2_ref,
          out_ref,
          m_ref, s_ref, acc_ref, max_ref,
          *, nb, nblocks):
    pid = pl.program_id(0)

    @pl.when(pid == 0)
    def _init():
        m_ref[...] = jnp.full_like(m_ref[...], -jnp.inf)
        s_ref[...] = jnp.zeros_like(s_ref[...])
        acc_ref[...] = jnp.zeros_like(acc_ref[...])
        max_ref[...] = jnp.full_like(max_ref[...], -jnp.inf)

    x = x_ref[...]
    base = pid * nb
    ri = lax.broadcasted_iota(jnp.int32, (nb, _B), 0) + base
    starts_row = starts_v_ref[...]
    ends_row = ends_v_ref[...]
    onehot_b = (ri >= starts_row) & (ri < ends_row)
    onehot_f = onehot_b.astype(jnp.float32)

    logits = lax.dot_general(x, w_att_ref[...], (((1,), (1,)), ((), ())),
                             preferred_element_type=jnp.float32)
    logits = logits + b_att_ref[...]
    scores = lax.dot_general(jnp.tanh(logits), ctx_ref[...],
                             (((1,), (1,)), ((), ())),
                             preferred_element_type=jnp.float32)

    m_old = m_ref[...]
    m_blk = jnp.max(jnp.where(onehot_b, scores, -jnp.inf), axis=0,
                    keepdims=True)
    m_new = jnp.maximum(m_old, m_blk)
    scale = jnp.where(m_new == -jnp.inf, 0.0, jnp.exp(m_old - m_new))
    p = jnp.where(onehot_b, jnp.exp(scores - m_new), 0.0)

    m_ref[...] = m_new
    s_ref[...] = s_ref[...] * scale + jnp.sum(p, axis=0, keepdims=True)
    q = jnp.concatenate([p, onehot_f], axis=1)
    scale2 = jnp.concatenate([scale, jnp.ones_like(scale)], axis=1)
    acc_ref[...] = acc_ref[...] * scale2 + lax.dot_general(
        x, q, (((0,), (0,)), ((), ())), preferred_element_type=jnp.float32)

    ri1 = lax.broadcasted_iota(jnp.int32, (nb, 1), 0) + base

    def seg_body(s, carry):
        st = starts_ref[s]
        en = starts_ref[s + 1]
        msk = (ri1 >= st) & (ri1 < en)
        colmax = jnp.max(jnp.where(msk, x, -jnp.inf), axis=0, keepdims=True)
        max_ref[pl.ds(s, 1), :] = jnp.maximum(max_ref[pl.ds(s, 1), :], colmax)
        return carry

    lax.fori_loop(lo_ref[pid], hi_ref[pid] + 1, seg_body, 0)

    @pl.when(pid == nblocks - 1)
    def _epilogue():
        counts = (ends_row - starts_row).astype(jnp.float32)
        acc = acc_ref[...]
        mean_t = acc[:, _B:] / jnp.maximum(counts, 1.0)
        graph_t = acc[:, :_B] / (s_ref[...] + 1e-8)
        attn = lax.dot_general(graph_t, w_out_ref[...],
                               (((0,), (1,)), ((), ())),
                               preferred_element_type=jnp.float32)
        attn = attn + b_out_ref[...]
        maxp = max_ref[...]
        w_s1 = w_s1_ref[...]
        h = lax.dot_general(mean_t, w_s1[:, 0:_D],
                            (((0,), (1,)), ((), ())),
                            preferred_element_type=jnp.float32)
        h = h + lax.dot_general(maxp, w_s1[:, _D:2 * _D],
                                (((1,), (1,)), ((), ())),
                                preferred_element_type=jnp.float32)
        h = h + lax.dot_general(attn, w_s1[:, 2 * _D:3 * _D],
                                (((1,), (1,)), ((), ())),
                                preferred_element_type=jnp.float32)
        h = jnp.maximum(h + b_s1_ref[...], 0.0)
        lg = lax.dot_general(h, w_s2_ref[...], (((1,), (1,)), ((), ())),
                             preferred_element_type=jnp.float32)
        lg = lg + b_s2_ref[...]
        lg = lg - jnp.max(lg, axis=1, keepdims=True)
        e = jnp.exp(lg)
        sel = e / jnp.sum(e, axis=1, keepdims=True)
        eye = (lax.broadcasted_iota(jnp.int32, (_D, _D), 0) ==
               lax.broadcasted_iota(jnp.int32, (_D, _D), 1)
               ).astype(jnp.float32)
        mean_bd = lax.dot_general(mean_t, eye, (((0,), (0,)), ((), ())),
                                  preferred_element_type=jnp.float32)
        out_ref[...] = (sel[:, 0:1] * mean_bd + sel[:, 1:2] * maxp +
                        sel[:, 2:3] * attn)


def kernel(x, batch, W_att, b_att, ctx, W_out, b_out, W_s1, b_s1, W_s2,
           b_s2):
    n, d = x.shape
    nb = _NB
    nblocks = n // nb
    assert n % nb == 0

    batch32 = batch.astype(jnp.int32)
    seg_ids = jnp.arange(_B + 1, dtype=jnp.int32)
    starts_ext = jnp.searchsorted(batch32, seg_ids, side="left").astype(
        jnp.int32)
    starts_v = starts_ext[:_B].reshape(1, _B)
    ends_v = starts_ext[1:].reshape(1, _B)
    blk0 = jnp.arange(nblocks, dtype=jnp.int32) * nb
    lo_arr = batch32[blk0]
    hi_arr = batch32[blk0 + (nb - 1)]

    body = functools.partial(_body, nb=nb, nblocks=nblocks)

    grid_spec = pltpu.PrefetchScalarGridSpec(
        num_scalar_prefetch=3,
        grid=(nblocks,),
        in_specs=[
            pl.BlockSpec((nb, d), lambda i, *_: (i, 0)),
            pl.BlockSpec((1, _B), lambda i, *_: (0, 0)),
            pl.BlockSpec((1, _B), lambda i, *_: (0, 0)),
            pl.BlockSpec(W_att.shape, lambda i, *_: (0, 0)),
            pl.BlockSpec((1, _B), lambda i, *_: (0, 0)),
            pl.BlockSpec((1, _B), lambda i, *_: (0, 0)),
            pl.BlockSpec(W_out.shape, lambda i, *_: (0, 0)),
            pl.BlockSpec((1, d), lambda i, *_: (0, 0)),
            pl.BlockSpec(W_s1.shape, lambda i, *_: (0, 0)),
            pl.BlockSpec((1, d), lambda i, *_: (0, 0)),
            pl.BlockSpec(W_s2.shape, lambda i, *_: (0, 0)),
            pl.BlockSpec((1, _S), lambda i, *_: (0, 0)),
        ],
        out_specs=pl.BlockSpec((_B, d), lambda i, *_: (0, 0)),
        scratch_shapes=[
            pltpu.VMEM((1, _B), jnp.float32),
            pltpu.VMEM((1, _B), jnp.float32),
            pltpu.VMEM((d, 2 * _B), jnp.float32),
            pltpu.VMEM((_B, d), jnp.float32),
        ],
    )

    return pl.pallas_call(
        body,
        grid_spec=grid_spec,
        out_shape=jax.ShapeDtypeStruct((_B, d), jnp.float32),
        compiler_params=pltpu.CompilerParams(
            dimension_semantics=("arbitrary",)),
    )(starts_ext, lo_arr, hi_arr,
      x, starts_v, ends_v, W_att, b_att.reshape(1, -1), ctx.reshape(1, -1),
      W_out, b_out.reshape(1, -1), W_s1, b_s1.reshape(1, -1), W_s2,
      b_s2.reshape(1, -1))
